# Initial kernel scaffold; baseline (speedup 1.0000x reference)
import jax
import jax.numpy as jnp
from jax.experimental import pallas as pl


def kernel(pred, target):
    raise NotImplementedError("write your pallas kernel here")


# speedup vs baseline: 20.4654x; 20.4654x over previous
import jax
import jax.numpy as jnp
from jax.experimental import pallas as pl
from jax.experimental.pallas import tpu as pltpu

B, N, D = 64, 256, 512
BB = 8


def _lap_loss_kernel(pred_ref, target_ref, out_ref):
    i = pl.program_id(0)
    e = pred_ref[...] - target_ref[...]
    up = jnp.concatenate([e[:, 1:, :], e[:, :1, :]], axis=1)
    dn = jnp.concatenate([e[:, -1:, :], e[:, :-1, :]], axis=1)
    s = jnp.sum(jnp.abs(e - 0.5 * (up + dn)))

    @pl.when(i == 0)
    def _init():
        out_ref[...] = jnp.zeros_like(out_ref)

    out_ref[...] += jnp.reshape(s, (1, 1))


def kernel(pred, target):
    grid = (B // BB,)
    total = pl.pallas_call(
        _lap_loss_kernel,
        grid=grid,
        in_specs=[
            pl.BlockSpec((BB, N, D), lambda i: (i, 0, 0)),
            pl.BlockSpec((BB, N, D), lambda i: (i, 0, 0)),
        ],
        out_specs=pl.BlockSpec((1, 1), lambda i: (0, 0)),
        out_shape=jax.ShapeDtypeStruct((1, 1), jnp.float32),
    )(pred, target)
    return total[0, 0] / (B * N * D)
